# Initial kernel scaffold; baseline (speedup 1.0000x reference)
import jax
import jax.numpy as jnp
from jax.experimental import pallas as pl


def kernel(x, v, z, pos, batch, W1, b1, W2, b2):
    raise NotImplementedError("write your pallas kernel here")


# speedup vs baseline: 4028.9312x; 4028.9312x over previous
import jax
import jax.numpy as jnp
from jax.experimental import pallas as pl
from jax.experimental.pallas import tpu as pltpu

_HIDDEN = 128
_NB = 256
_RC = 4.6
_FACTOR = 0.5 * 27.211386024367243 * 0.5291772105638411
_WSUM = 1.875
_INV036 = 1.0 / 0.36787944117144233


def _band_kernel(bounds_ref, h_ref, qw_ref, pos_ref, bcol_ref,
                 qT_ref, posT_ref, brow_ref,
                 W1_ref, b1_ref, W2_ref, b2_ref, out_ref):
    b = pl.program_id(0)
    lo = bounds_ref[b, 0]
    hi = bounds_ref[b, 1]
    pi = pos_ref[...]
    qwi = qw_ref[...]
    bi = bcol_ref[...]
    pn_i = jnp.sum(pi * pi, axis=1, keepdims=True)
    row_ids = b * _NB + jax.lax.broadcasted_iota(jnp.int32, (_NB, 1), 0)
    inv_rc = 1.0 / _RC
    scale = _FACTOR / _WSUM

    def body(j, acc):
        js = pl.multiple_of(j * _NB, _NB)
        pj = posT_ref[:, pl.ds(js, _NB)]
        qj = qT_ref[:, pl.ds(js, _NB)]
        bj = brow_ref[:, pl.ds(js, _NB)]
        pn_j = jnp.sum(pj * pj, axis=0, keepdims=True)
        dot = jnp.dot(pi, pj, preferred_element_type=jnp.float32)
        d2 = jnp.maximum(pn_i + pn_j - 2.0 * dot, 1e-12)
        rsq = jax.lax.rsqrt(d2)
        d = d2 * rsq
        t = jnp.minimum(d * inv_rc, 1.0 - 1e-6)
        t2 = t * t
        fc = 1.0 - jnp.exp(-1.0 / (1.0 - t2)) * _INV036
        qq = jnp.dot(qwi, qj, preferred_element_type=jnp.float32)
        col_ids = js + jax.lax.broadcasted_iota(jnp.int32, (1, _NB), 1)
        mask = (bi == bj) & (row_ids != col_ids)
        e = jnp.where(mask, scale * fc * qq * rsq, 0.0)
        return acc + jnp.sum(e, axis=1, keepdims=True)

    acc = jax.lax.fori_loop(lo, hi + 1, body,
                            jnp.zeros((_NB, 1), jnp.float32))

    hmid = jnp.dot(h_ref[...], W1_ref[...],
                   preferred_element_type=jnp.float32) + b1_ref[...]
    hmid = hmid * jax.nn.sigmoid(hmid)
    out = jnp.dot(hmid, W2_ref[...],
                  preferred_element_type=jnp.float32) + b2_ref[...]
    out_ref[...] = out + acc


def kernel(x, v, z, pos, batch, W1, b1, W2, b2):
    N = x.shape[0]
    nb = N // _NB
    h = x[:, :_HIDDEN]
    q = x[:, _HIDDEN:]
    qweights = jnp.array([1.0, 0.5, 0.25, 0.125], dtype=x.dtype)
    qw8 = jnp.pad(q * qweights, ((0, 0), (0, 4)))
    qT8 = jnp.pad(q, ((0, 0), (0, 4))).T
    pos8 = jnp.pad(pos, ((0, 0), (0, 5)))
    posT8 = pos8.T
    bcol = batch[:, None]
    brow = batch[None, :]

    b_first = batch[::_NB]
    b_last = batch[_NB - 1::_NB]
    lo_atom = jnp.searchsorted(batch, b_first, side='left')
    hi_atom = jnp.searchsorted(batch, b_last, side='right') - 1
    bounds = jnp.stack([lo_atom // _NB, hi_atom // _NB],
                       axis=1).astype(jnp.int32)

    grid_spec = pltpu.PrefetchScalarGridSpec(
        num_scalar_prefetch=1,
        grid=(nb,),
        in_specs=[
            pl.BlockSpec((_NB, _HIDDEN), lambda b, *_: (b, 0)),
            pl.BlockSpec((_NB, 8), lambda b, *_: (b, 0)),
            pl.BlockSpec((_NB, 8), lambda b, *_: (b, 0)),
            pl.BlockSpec((_NB, 1), lambda b, *_: (b, 0)),
            pl.BlockSpec((8, N), lambda b, *_: (0, 0)),
            pl.BlockSpec((8, N), lambda b, *_: (0, 0)),
            pl.BlockSpec((1, N), lambda b, *_: (0, 0)),
            pl.BlockSpec((_HIDDEN, _HIDDEN // 2), lambda b, *_: (0, 0)),
            pl.BlockSpec((1, _HIDDEN // 2), lambda b, *_: (0, 0)),
            pl.BlockSpec((_HIDDEN // 2, 1), lambda b, *_: (0, 0)),
            pl.BlockSpec((1, 1), lambda b, *_: (0, 0)),
        ],
        out_specs=pl.BlockSpec((_NB, 1), lambda b, *_: (b, 0)),
    )

    out = pl.pallas_call(
        _band_kernel,
        grid_spec=grid_spec,
        out_shape=jax.ShapeDtypeStruct((N, 1), jnp.float32),
    )(bounds, h, qw8, pos8, bcol, qT8, posT8, brow,
      W1, b1[None, :], W2, b2[None, :])
    return out
